# baseline (speedup 1.0000x reference)
import functools

import jax
import jax.numpy as jnp
from jax import lax
from jax.experimental import pallas as pl
from jax.experimental.pallas import tpu as pltpu
from jax.experimental.pallas import tpu_sc as plsc

N = 10000
F = 128
FH = 64
NP = 10240
E = 320000
EP = 327680
BK = 128
NBLK = 80
NBLKC = 160
EPT = NBLK * BK
RPT = NP // 16

_mesh = plsc.VectorSubcoreMesh(core_axis_name="c", subcore_axis_name="s")


def _zero_fill(ref, nwords):
    def body(i, _):
        ref[pl.ds(i * 16, 16)] = jnp.zeros((16,), jnp.float32)
        return 0
    lax.fori_loop(0, nwords // 16, body, 0)


@functools.partial(
    pl.kernel,
    out_type=jax.ShapeDtypeStruct((32, NP), jnp.float32),
    mesh=_mesh,
    compiler_params=pltpu.CompilerParams(needs_layout_passes=False),
    scratch_types=[
        pltpu.VMEM((EPT,), jnp.int32),
        pltpu.VMEM((NP,), jnp.float32),
    ],
)
def _deg_kernel(dst_hbm, out_hbm, dst_v, deg_v):
    c = lax.axis_index("c")
    s = lax.axis_index("s")
    chunk = c * 16 + s
    _zero_fill(deg_v, NP)
    pltpu.sync_copy(dst_hbm.at[chunk], dst_v)
    ones = jnp.ones((16,), jnp.float32)

    def body(i, _):
        idx = dst_v[pl.ds(i * 16, 16)]
        plsc.addupdate_scatter(deg_v, [idx], ones)
        return 0
    lax.fori_loop(0, EPT // 16, body, 0)
    pltpu.sync_copy(deg_v, out_hbm.at[chunk])


SUP = 8
CH = NBLKC // SUP


@functools.partial(
    pl.kernel,
    out_type=jax.ShapeDtypeStruct((2, NP, FH), jnp.float32),
    mesh=_mesh,
    compiler_params=pltpu.CompilerParams(use_tc_tiling_on_sc=False),
    scratch_types=[
        [pltpu.VMEM((CH, BK), jnp.int32) for _ in range(2)],
        [pltpu.VMEM((CH, BK), jnp.int32) for _ in range(2)],
        [pltpu.VMEM((BK, FH), jnp.float32) for _ in range(4)],
        [pltpu.SemaphoreType.DMA for _ in range(4)],
        [pltpu.SemaphoreType.DMA for _ in range(4)],
        pltpu.SemaphoreType.DMA,
        pltpu.VMEM_SHARED((NP, FH), jnp.float32),
        pltpu.VMEM_SHARED((NP, FH), jnp.float32),
    ],
)
def _agg_kernel(y_hbm, src_hbm, dst_hbm, out_hbm, srcs, dsts, bufs, gs, ss,
                isem, y_sh, agg_sh):
    c = lax.axis_index("c")
    s = lax.axis_index("s")
    b0 = bufs[0]

    def init(i, _):
        r = s * RPT + i * BK
        pltpu.sync_copy(y_hbm.at[c, pl.ds(r, BK)], b0)
        pltpu.sync_copy(b0, y_sh.at[pl.ds(r, BK)])
        pltpu.sync_copy(b0, agg_sh.at[pl.ds(r, BK)])
        return 0
    lax.fori_loop(0, RPT // BK, init, 0)
    pltpu.sync_copy(src_hbm.at[s, pl.ds(0, CH)], srcs[0])
    pltpu.sync_copy(dst_hbm.at[s, pl.ds(0, CH)], dsts[0])
    plsc.subcore_barrier()

    for sup in range(SUP):
        src_v = srcs[sup % 2]
        dst_v = dsts[sup % 2]
        if sup < SUP - 1:
            pltpu.async_copy(src_hbm.at[s, pl.ds((sup + 1) * CH, CH)],
                             srcs[(sup + 1) % 2], isem)
            pltpu.async_copy(dst_hbm.at[s, pl.ds((sup + 1) * CH, CH)],
                             dsts[(sup + 1) % 2], isem)
        for k in range(4):
            pltpu.async_copy(y_sh.at[src_v.at[k]], bufs[k], gs[k])

        def body(gg, _):
            g0 = gg * 4
            for k in range(4):
                pltpu.make_async_copy(y_sh.at[src_v.at[g0 + k]],
                                      bufs[k], gs[k]).wait()
                pltpu.async_copy(bufs[k], agg_sh.at[dst_v.at[g0 + k]], ss[k],
                                 add=True)
            for k in range(4):
                @pl.when(g0 + 4 + k < CH)
                def _():
                    pltpu.make_async_copy(bufs[k],
                                          agg_sh.at[dst_v.at[g0 + k]],
                                          ss[k]).wait()
                    pltpu.async_copy(y_sh.at[src_v.at[g0 + 4 + k]],
                                     bufs[k], gs[k])
            return 0
        lax.fori_loop(0, CH // 4, body, 0)
        for k in range(4):
            pltpu.make_async_copy(bufs[k], agg_sh.at[dst_v.at[CH - 4 + k]],
                                  ss[k]).wait()
        if sup < SUP - 1:
            pltpu.make_async_copy(src_hbm.at[s, pl.ds((sup + 1) * CH, CH)],
                                  srcs[(sup + 1) % 2], isem).wait()
            pltpu.make_async_copy(dst_hbm.at[s, pl.ds((sup + 1) * CH, CH)],
                                  dsts[(sup + 1) % 2], isem).wait()
    plsc.subcore_barrier()

    def copy_out(i, _):
        r = s * RPT + i * BK
        pltpu.sync_copy(agg_sh.at[pl.ds(r, BK)], b0)
        pltpu.sync_copy(b0, out_hbm.at[c, pl.ds(r, BK)])
        return 0
    lax.fori_loop(0, RPT // BK, copy_out, 0)


@functools.partial(
    pl.kernel,
    out_type=jax.ShapeDtypeStruct((32, NP), jnp.float32),
    mesh=_mesh,
    compiler_params=pltpu.CompilerParams(needs_layout_passes=False),
    scratch_types=[
        pltpu.VMEM((NP,), jnp.float32),
        pltpu.VMEM((EPT,), jnp.int32),
        pltpu.VMEM((EPT,), jnp.int32),
        pltpu.VMEM((NP,), jnp.float32),
    ],
)
def _l2_kernel(t_hbm, srcf_hbm, dstf_hbm, out_hbm, t_v, src_v, dst_v, acc_v):
    c = lax.axis_index("c")
    s = lax.axis_index("s")
    chunk = c * 16 + s
    _zero_fill(acc_v, NP)
    pltpu.sync_copy(t_hbm, t_v)
    pltpu.sync_copy(srcf_hbm.at[chunk], src_v)
    pltpu.sync_copy(dstf_hbm.at[chunk], dst_v)

    def body(i, _):
        sidx = src_v[pl.ds(i * 16, 16)]
        didx = dst_v[pl.ds(i * 16, 16)]
        vals = plsc.load_gather(t_v, [sidx])
        plsc.addupdate_scatter(acc_v, [didx], vals)
        return 0
    lax.fori_loop(0, EPT // 16, body, 0)
    pltpu.sync_copy(acc_v, out_hbm.at[chunk])


_BR = 1024


def _mmxw_body(x_ref, w_ref, xw_ref):
    xw_ref[...] = jnp.dot(x_ref[...], w_ref[...],
                          preferred_element_type=jnp.float32)


def _mmxw(x_p, w1):
    return pl.pallas_call(
        _mmxw_body,
        grid=(NP // _BR,),
        in_specs=[
            pl.BlockSpec((_BR, F), lambda i: (i, 0)),
            pl.BlockSpec((F, F), lambda i: (0, 0)),
        ],
        out_specs=pl.BlockSpec((_BR, F), lambda i: (i, 0)),
        out_shape=jax.ShapeDtypeStruct((NP, F), jnp.float32),
    )(x_p, w1)


def _scale_body(xw_ref, parts_ref, y_ref, dinv_ref):
    deg = jnp.sum(parts_ref[...], axis=0) + 1.0
    dinv = lax.rsqrt(deg)
    dinv_ref[...] = dinv
    ys = xw_ref[...] * dinv
    y_ref[...] = jnp.stack([ys[:, :FH], ys[:, FH:]])


def _scale(xw, parts3):
    return pl.pallas_call(
        _scale_body,
        grid=(NP // _BR,),
        in_specs=[
            pl.BlockSpec((_BR, F), lambda i: (i, 0)),
            pl.BlockSpec((32, _BR, 1), lambda i: (0, i, 0)),
        ],
        out_specs=[
            pl.BlockSpec((2, _BR, FH), lambda i: (0, i, 0)),
            pl.BlockSpec((_BR, 1), lambda i: (i, 0)),
        ],
        out_shape=[
            jax.ShapeDtypeStruct((2, NP, FH), jnp.float32),
            jax.ShapeDtypeStruct((NP, 1), jnp.float32),
        ],
    )(xw, parts3)


def _mm2_body(p_ref, dinv_ref, b1_ref, w2_ref, t_ref):
    p = p_ref[...]
    dinv = dinv_ref[...]
    h0 = jnp.maximum(p[0] * dinv + b1_ref[:, :FH], 0.0)
    h1 = jnp.maximum(p[1] * dinv + b1_ref[:, FH:], 0.0)
    w2 = w2_ref[...]
    sc = (jnp.dot(h0, w2[:FH], preferred_element_type=jnp.float32)
          + jnp.dot(h1, w2[FH:], preferred_element_type=jnp.float32))
    t_ref[...] = sc * dinv


def _mm2(parts, dinv, b1, w2):
    return pl.pallas_call(
        _mm2_body,
        grid=(NP // _BR,),
        in_specs=[
            pl.BlockSpec((2, _BR, FH), lambda i: (0, i, 0)),
            pl.BlockSpec((_BR, 1), lambda i: (i, 0)),
            pl.BlockSpec((1, F), lambda i: (0, 0)),
            pl.BlockSpec((F, 1), lambda i: (0, 0)),
        ],
        out_specs=pl.BlockSpec((_BR, 1), lambda i: (i, 0)),
        out_shape=jax.ShapeDtypeStruct((NP, 1), jnp.float32),
    )(parts, dinv, b1, w2)


def kernel(x, edge_index, W1, b1, W2, b2):
    src = edge_index[0].astype(jnp.int32)
    dst = edge_index[1].astype(jnp.int32)
    pad = EP - E
    src_p = jnp.concatenate([src, jnp.zeros((pad,), jnp.int32)])
    dst_p = jnp.concatenate([dst, jnp.full((pad,), NP - 1, jnp.int32)])
    srcf = src_p.reshape(32, EPT)
    dstf = dst_p.reshape(32, EPT)
    src16 = src_p.reshape(16, NBLKC, BK)
    dstC = dst_p.reshape(16, NBLKC, BK)
    x_p = jnp.pad(x, ((0, NP - N), (0, 0)))

    deg_parts = _deg_kernel(dstf)
    xw = _mmxw(x_p, W1)
    parts3 = deg_parts.reshape(32, NP, 1)
    y_split, dinv = _scale(xw, parts3)
    agg_parts = _agg_kernel(y_split, src16, dstC)
    t = _mm2(agg_parts, dinv, b1.reshape(1, F), W2)
    l2_parts = _l2_kernel(t.reshape(NP), srcf, dstf)

    out = (jnp.sum(l2_parts, axis=0) + t[:, 0]) * dinv[:, 0] + b2[0]
    return out[:N]


# speedup vs baseline: 1.0016x; 1.0016x over previous
import functools

import jax
import jax.numpy as jnp
from jax import lax
from jax.experimental import pallas as pl
from jax.experimental.pallas import tpu as pltpu
from jax.experimental.pallas import tpu_sc as plsc

N = 10000
F = 128
FH = 64
NP = 10240
E = 320000
EP = 327680
BK = 128
NBLK = 80
NBLKC = 160
EPT = NBLK * BK
RPT = NP // 16

_mesh = plsc.VectorSubcoreMesh(core_axis_name="c", subcore_axis_name="s")


def _zero_fill(ref, nwords):
    def body(i, _):
        ref[pl.ds(i * 16, 16)] = jnp.zeros((16,), jnp.float32)
        return 0
    lax.fori_loop(0, nwords // 16, body, 0)


@functools.partial(
    pl.kernel,
    out_type=jax.ShapeDtypeStruct((32, NP), jnp.float32),
    mesh=_mesh,
    compiler_params=pltpu.CompilerParams(needs_layout_passes=False),
    scratch_types=[
        pltpu.VMEM((EPT,), jnp.int32),
        pltpu.VMEM((NP,), jnp.float32),
    ],
)
def _deg_kernel(dst_hbm, out_hbm, dst_v, deg_v):
    c = lax.axis_index("c")
    s = lax.axis_index("s")
    chunk = c * 16 + s
    _zero_fill(deg_v, NP)
    pltpu.sync_copy(dst_hbm.at[chunk], dst_v)
    ones = jnp.ones((16,), jnp.float32)

    def body(i, _):
        idx = dst_v[pl.ds(i * 16, 16)]
        plsc.addupdate_scatter(deg_v, [idx], ones)
        return 0
    lax.fori_loop(0, EPT // 16, body, 0)
    pltpu.sync_copy(deg_v, out_hbm.at[chunk])


SUP = 8
CH = NBLKC // SUP


@functools.partial(
    pl.kernel,
    out_type=jax.ShapeDtypeStruct((2, NP, FH), jnp.float32),
    mesh=_mesh,
    compiler_params=pltpu.CompilerParams(use_tc_tiling_on_sc=False),
    scratch_types=[
        [pltpu.VMEM((CH, BK), jnp.int32) for _ in range(2)],
        [pltpu.VMEM((CH, BK), jnp.int32) for _ in range(2)],
        [pltpu.VMEM((BK, FH), jnp.float32) for _ in range(4)],
        [pltpu.SemaphoreType.DMA for _ in range(4)],
        [pltpu.SemaphoreType.DMA for _ in range(4)],
        pltpu.SemaphoreType.DMA,
        pltpu.VMEM_SHARED((NP, FH), jnp.float32),
        pltpu.VMEM_SHARED((NP, FH), jnp.float32),
    ],
)
def _agg_kernel(y_hbm, src_hbm, dst_hbm, out_hbm, srcs, dsts, bufs, gs, ss,
                isem, y_sh, agg_sh):
    c = lax.axis_index("c")
    s = lax.axis_index("s")
    b0 = bufs[0]

    def init(i, _):
        r = s * RPT + i * BK
        pltpu.sync_copy(y_hbm.at[c, pl.ds(r, BK)], b0)
        pltpu.sync_copy(b0, y_sh.at[pl.ds(r, BK)])
        pltpu.sync_copy(b0, agg_sh.at[pl.ds(r, BK)])
        return 0
    lax.fori_loop(0, RPT // BK, init, 0)
    pltpu.sync_copy(src_hbm.at[s, pl.ds(0, CH)], srcs[0])
    pltpu.sync_copy(dst_hbm.at[s, pl.ds(0, CH)], dsts[0])
    plsc.subcore_barrier()

    for sup in range(SUP):
        src_v = srcs[sup % 2]
        dst_v = dsts[sup % 2]
        if sup < SUP - 1:
            pltpu.async_copy(src_hbm.at[s, pl.ds((sup + 1) * CH, CH)],
                             srcs[(sup + 1) % 2], isem)
            pltpu.async_copy(dst_hbm.at[s, pl.ds((sup + 1) * CH, CH)],
                             dsts[(sup + 1) % 2], isem)
        for k in range(4):
            pltpu.async_copy(y_sh.at[src_v.at[k]], bufs[k], gs[k])

        def body(gg, _):
            g0 = gg * 4
            for k in range(4):
                pltpu.make_async_copy(y_sh.at[src_v.at[g0 + k]],
                                      bufs[k], gs[k]).wait()
                pltpu.async_copy(bufs[k], agg_sh.at[dst_v.at[g0 + k]], ss[k],
                                 add=True)
            for k in range(4):
                @pl.when(g0 + 4 + k < CH)
                def _():
                    pltpu.make_async_copy(bufs[k],
                                          agg_sh.at[dst_v.at[g0 + k]],
                                          ss[k]).wait()
                    pltpu.async_copy(y_sh.at[src_v.at[g0 + 4 + k]],
                                     bufs[k], gs[k])
            return 0
        lax.fori_loop(0, CH // 4, body, 0)
        for k in range(4):
            pltpu.make_async_copy(bufs[k], agg_sh.at[dst_v.at[CH - 4 + k]],
                                  ss[k]).wait()
        if sup < SUP - 1:
            pltpu.make_async_copy(src_hbm.at[s, pl.ds((sup + 1) * CH, CH)],
                                  srcs[(sup + 1) % 2], isem).wait()
            pltpu.make_async_copy(dst_hbm.at[s, pl.ds((sup + 1) * CH, CH)],
                                  dsts[(sup + 1) % 2], isem).wait()
    plsc.subcore_barrier()

    def copy_out(i, _):
        r = s * RPT + i * BK
        pltpu.sync_copy(agg_sh.at[pl.ds(r, BK)], b0)
        pltpu.sync_copy(b0, out_hbm.at[c, pl.ds(r, BK)])
        return 0
    lax.fori_loop(0, RPT // BK, copy_out, 0)


@functools.partial(
    pl.kernel,
    out_type=jax.ShapeDtypeStruct((32, NP), jnp.float32),
    mesh=_mesh,
    compiler_params=pltpu.CompilerParams(needs_layout_passes=False),
    scratch_types=[
        pltpu.VMEM((NP,), jnp.float32),
        pltpu.VMEM((EPT,), jnp.int32),
        pltpu.VMEM((EPT,), jnp.int32),
        pltpu.VMEM((NP,), jnp.float32),
    ],
)
def _l2_kernel(t_hbm, srcf_hbm, dstf_hbm, out_hbm, t_v, src_v, dst_v, acc_v):
    c = lax.axis_index("c")
    s = lax.axis_index("s")
    chunk = c * 16 + s
    _zero_fill(acc_v, NP)
    pltpu.sync_copy(t_hbm, t_v)
    pltpu.sync_copy(srcf_hbm.at[chunk], src_v)
    pltpu.sync_copy(dstf_hbm.at[chunk], dst_v)

    def body(i, _):
        sidx = src_v[pl.ds(i * 16, 16)]
        didx = dst_v[pl.ds(i * 16, 16)]
        vals = plsc.load_gather(t_v, [sidx])
        plsc.addupdate_scatter(acc_v, [didx], vals)
        return 0
    lax.fori_loop(0, EPT // 16, body, 0)
    pltpu.sync_copy(acc_v, out_hbm.at[chunk])


_BR = 1024


def _mm1_body(x_ref, w_ref, parts_ref, y_ref, dinv_ref):
    xw = jnp.dot(x_ref[...], w_ref[...], preferred_element_type=jnp.float32)
    deg = jnp.sum(parts_ref[...], axis=0) + 1.0
    dinv = lax.rsqrt(deg)
    dinv_ref[...] = dinv
    ys = xw * dinv
    y_ref[...] = jnp.stack([ys[:, :FH], ys[:, FH:]])


def _mm1(x_p, w1, parts3):
    return pl.pallas_call(
        _mm1_body,
        grid=(NP // _BR,),
        in_specs=[
            pl.BlockSpec((_BR, F), lambda i: (i, 0)),
            pl.BlockSpec((F, F), lambda i: (0, 0)),
            pl.BlockSpec((32, _BR, 1), lambda i: (0, i, 0)),
        ],
        out_specs=[
            pl.BlockSpec((2, _BR, FH), lambda i: (0, i, 0)),
            pl.BlockSpec((_BR, 1), lambda i: (i, 0)),
        ],
        out_shape=[
            jax.ShapeDtypeStruct((2, NP, FH), jnp.float32),
            jax.ShapeDtypeStruct((NP, 1), jnp.float32),
        ],
    )(x_p, w1, parts3)


def _mm2_body(p_ref, dinv_ref, b1_ref, w2_ref, t_ref):
    p = p_ref[...]
    dinv = dinv_ref[...]
    h0 = jnp.maximum(p[0] * dinv + b1_ref[:, :FH], 0.0)
    h1 = jnp.maximum(p[1] * dinv + b1_ref[:, FH:], 0.0)
    w2 = w2_ref[...]
    sc = (jnp.dot(h0, w2[:FH], preferred_element_type=jnp.float32)
          + jnp.dot(h1, w2[FH:], preferred_element_type=jnp.float32))
    t_ref[...] = sc * dinv


def _mm2(parts, dinv, b1, w2):
    return pl.pallas_call(
        _mm2_body,
        grid=(NP // _BR,),
        in_specs=[
            pl.BlockSpec((2, _BR, FH), lambda i: (0, i, 0)),
            pl.BlockSpec((_BR, 1), lambda i: (i, 0)),
            pl.BlockSpec((1, F), lambda i: (0, 0)),
            pl.BlockSpec((F, 1), lambda i: (0, 0)),
        ],
        out_specs=pl.BlockSpec((_BR, 1), lambda i: (i, 0)),
        out_shape=jax.ShapeDtypeStruct((NP, 1), jnp.float32),
    )(parts, dinv, b1, w2)


def kernel(x, edge_index, W1, b1, W2, b2):
    src = edge_index[0].astype(jnp.int32)
    dst = edge_index[1].astype(jnp.int32)
    pad = EP - E
    src_p = jnp.concatenate([src, jnp.zeros((pad,), jnp.int32)])
    dst_p = jnp.concatenate([dst, jnp.full((pad,), NP - 1, jnp.int32)])
    srcf = src_p.reshape(32, EPT)
    dstf = dst_p.reshape(32, EPT)
    src16 = src_p.reshape(16, NBLKC, BK)
    dstC = dst_p.reshape(16, NBLKC, BK)
    x_p = jnp.pad(x, ((0, NP - N), (0, 0)))

    deg_parts = _deg_kernel(dstf)
    parts3 = deg_parts.reshape(32, NP, 1)
    y_split, dinv = _mm1(x_p, W1, parts3)
    agg_parts = _agg_kernel(y_split, src16, dstC)
    t = _mm2(agg_parts, dinv, b1.reshape(1, F), W2)
    l2_parts = _l2_kernel(t.reshape(NP), srcf, dstf)

    out = (jnp.sum(l2_parts, axis=0) + t[:, 0]) * dinv[:, 0] + b2[0]
    return out[:N]


# speedup vs baseline: 1.6254x; 1.6229x over previous
import functools

import jax
import jax.numpy as jnp
from jax import lax
from jax.experimental import pallas as pl
from jax.experimental.pallas import tpu as pltpu
from jax.experimental.pallas import tpu_sc as plsc

N = 10000
F = 128
FH = 64
NP = 10240
E = 320000
EP = 327680
BK = 128
NBLK = 80
NBLKC = 160
EPT = NBLK * BK
RPT = NP // 16

_mesh = plsc.VectorSubcoreMesh(core_axis_name="c", subcore_axis_name="s")


def _zero_fill(ref, nwords):
    def body(i, _):
        ref[pl.ds(i * 16, 16)] = jnp.zeros((16,), jnp.float32)
        return 0
    lax.fori_loop(0, nwords // 16, body, 0)


@functools.partial(
    pl.kernel,
    out_type=jax.ShapeDtypeStruct((2, NP), jnp.float32),
    mesh=_mesh,
    scratch_types=[
        pltpu.VMEM((NBLK, BK), jnp.int32),
        pltpu.VMEM((BK,), jnp.float32),
        pltpu.VMEM((RPT,), jnp.float32),
        pltpu.VMEM_SHARED((NP,), jnp.float32),
        pltpu.SemaphoreType.DMA,
    ],
)
def _deg_kernel(dst_hbm, out_hbm, dst_v, ones_v, buf_v, deg_sh, sem):
    c = lax.axis_index("c")
    s = lax.axis_index("s")
    chunk = c * 16 + s
    _zero_fill(buf_v, RPT)
    pltpu.sync_copy(buf_v, deg_sh.at[pl.ds(s * RPT, RPT)])

    def fill_ones(i, _):
        ones_v[pl.ds(i * 16, 16)] = jnp.ones((16,), jnp.float32)
        return 0
    lax.fori_loop(0, BK // 16, fill_ones, 0)
    pltpu.sync_copy(dst_hbm.at[chunk], dst_v)
    plsc.subcore_barrier()

    def body(j, _):
        for k in range(8):
            pltpu.async_copy(ones_v, deg_sh.at[dst_v.at[j * 8 + k]], sem,
                             add=True)
        for k in range(8):
            pltpu.make_async_copy(ones_v, deg_sh.at[dst_v.at[j * 8 + k]],
                                  sem).wait()
        return 0
    lax.fori_loop(0, NBLK // 8, body, 0)
    plsc.subcore_barrier()
    pltpu.sync_copy(deg_sh.at[pl.ds(s * RPT, RPT)], buf_v)
    pltpu.sync_copy(buf_v, out_hbm.at[c, pl.ds(s * RPT, RPT)])


SUP = 8
CH = NBLKC // SUP


@functools.partial(
    pl.kernel,
    out_type=jax.ShapeDtypeStruct((2, NP, FH), jnp.float32),
    mesh=_mesh,
    compiler_params=pltpu.CompilerParams(use_tc_tiling_on_sc=False),
    scratch_types=[
        [pltpu.VMEM((CH, BK), jnp.int32) for _ in range(2)],
        [pltpu.VMEM((CH, BK), jnp.int32) for _ in range(2)],
        [pltpu.VMEM((BK, FH), jnp.float32) for _ in range(4)],
        [pltpu.SemaphoreType.DMA for _ in range(4)],
        [pltpu.SemaphoreType.DMA for _ in range(4)],
        pltpu.SemaphoreType.DMA,
        pltpu.VMEM_SHARED((NP, FH), jnp.float32),
        pltpu.VMEM_SHARED((NP, FH), jnp.float32),
    ],
)
def _agg_kernel(y_hbm, src_hbm, dst_hbm, out_hbm, srcs, dsts, bufs, gs, ss,
                isem, y_sh, agg_sh):
    c = lax.axis_index("c")
    s = lax.axis_index("s")
    b0 = bufs[0]

    def init(i, _):
        r = s * RPT + i * BK
        pltpu.sync_copy(y_hbm.at[c, pl.ds(r, BK)], b0)
        pltpu.sync_copy(b0, y_sh.at[pl.ds(r, BK)])
        pltpu.sync_copy(b0, agg_sh.at[pl.ds(r, BK)])
        return 0
    lax.fori_loop(0, RPT // BK, init, 0)
    pltpu.sync_copy(src_hbm.at[s, pl.ds(0, CH)], srcs[0])
    pltpu.sync_copy(dst_hbm.at[s, pl.ds(0, CH)], dsts[0])
    plsc.subcore_barrier()

    for sup in range(SUP):
        src_v = srcs[sup % 2]
        dst_v = dsts[sup % 2]
        if sup < SUP - 1:
            pltpu.async_copy(src_hbm.at[s, pl.ds((sup + 1) * CH, CH)],
                             srcs[(sup + 1) % 2], isem)
            pltpu.async_copy(dst_hbm.at[s, pl.ds((sup + 1) * CH, CH)],
                             dsts[(sup + 1) % 2], isem)
        pltpu.async_copy(y_sh.at[src_v.at[0]], bufs[0], gs[0])
        pltpu.async_copy(y_sh.at[src_v.at[1]], bufs[1], gs[1])

        def body(gg, _):
            g0 = gg * 4
            for k in range(4):
                g = g0 + k
                pltpu.make_async_copy(y_sh.at[src_v.at[g]],
                                      bufs[k], gs[k]).wait()
                pltpu.async_copy(bufs[k], agg_sh.at[dst_v.at[g]], ss[k],
                                 add=True)
                kp = (k + 2) % 4

                @pl.when(g >= 2)
                def _():
                    pltpu.make_async_copy(bufs[kp],
                                          agg_sh.at[dst_v.at[g - 2]],
                                          ss[kp]).wait()

                @pl.when(g + 2 < CH)
                def _():
                    pltpu.async_copy(y_sh.at[src_v.at[g + 2]],
                                     bufs[kp], gs[kp])
            return 0
        lax.fori_loop(0, CH // 4, body, 0)
        for k in range(2):
            g = CH - 2 + k
            pltpu.make_async_copy(bufs[g % 4], agg_sh.at[dst_v.at[g]],
                                  ss[g % 4]).wait()
        if sup < SUP - 1:
            pltpu.make_async_copy(src_hbm.at[s, pl.ds((sup + 1) * CH, CH)],
                                  srcs[(sup + 1) % 2], isem).wait()
            pltpu.make_async_copy(dst_hbm.at[s, pl.ds((sup + 1) * CH, CH)],
                                  dsts[(sup + 1) % 2], isem).wait()
    plsc.subcore_barrier()

    def copy_out(i, _):
        r = s * RPT + i * BK
        pltpu.sync_copy(agg_sh.at[pl.ds(r, BK)], b0)
        pltpu.sync_copy(b0, out_hbm.at[c, pl.ds(r, BK)])
        return 0
    lax.fori_loop(0, RPT // BK, copy_out, 0)


@functools.partial(
    pl.kernel,
    out_type=jax.ShapeDtypeStruct((2, NP), jnp.float32),
    mesh=_mesh,
    compiler_params=pltpu.CompilerParams(needs_layout_passes=False),
    scratch_types=[
        pltpu.VMEM((NP,), jnp.float32),
        pltpu.VMEM((EPT,), jnp.int32),
        pltpu.VMEM((NBLK, BK), jnp.int32),
        pltpu.VMEM((EPT,), jnp.float32),
        pltpu.VMEM((RPT,), jnp.float32),
        pltpu.VMEM_SHARED((NP,), jnp.float32),
        pltpu.SemaphoreType.DMA,
    ],
)
def _l2_kernel(t_hbm, srcf_hbm, dst_hbm, out_hbm, t_v, src_v, dst_v, vals_v,
               buf_v, acc_sh, sem):
    c = lax.axis_index("c")
    s = lax.axis_index("s")
    chunk = c * 16 + s
    _zero_fill(buf_v, RPT)
    pltpu.sync_copy(buf_v, acc_sh.at[pl.ds(s * RPT, RPT)])
    pltpu.sync_copy(t_hbm, t_v)
    pltpu.sync_copy(srcf_hbm.at[chunk], src_v)
    pltpu.sync_copy(dst_hbm.at[chunk], dst_v)

    def gather(i, _):
        idx = src_v[pl.ds(i * 16, 16)]
        vals_v[pl.ds(i * 16, 16)] = plsc.load_gather(t_v, [idx])
        return 0
    lax.fori_loop(0, EPT // 16, gather, 0)
    plsc.subcore_barrier()

    def scatter(j, _):
        for k in range(8):
            jj = j * 8 + k
            pltpu.async_copy(vals_v.at[pl.ds(jj * BK, BK)],
                             acc_sh.at[dst_v.at[jj]], sem, add=True)
        for k in range(8):
            jj = j * 8 + k
            pltpu.make_async_copy(vals_v.at[pl.ds(jj * BK, BK)],
                                  acc_sh.at[dst_v.at[jj]], sem).wait()
        return 0
    lax.fori_loop(0, NBLK // 8, scatter, 0)
    plsc.subcore_barrier()
    pltpu.sync_copy(acc_sh.at[pl.ds(s * RPT, RPT)], buf_v)
    pltpu.sync_copy(buf_v, out_hbm.at[c, pl.ds(s * RPT, RPT)])


_BR = 1024


def _mm1_body(x_ref, w_ref, parts_ref, y_ref, dinv_ref):
    xw = jnp.dot(x_ref[...], w_ref[...], preferred_element_type=jnp.float32)
    deg = parts_ref[0] + parts_ref[1] + 1.0
    dinv = lax.rsqrt(deg)
    dinv_ref[...] = dinv
    ys = xw * dinv
    y_ref[...] = jnp.stack([ys[:, :FH], ys[:, FH:]])


def _mm1(x_p, w1, parts3):
    return pl.pallas_call(
        _mm1_body,
        grid=(NP // _BR,),
        in_specs=[
            pl.BlockSpec((_BR, F), lambda i: (i, 0)),
            pl.BlockSpec((F, F), lambda i: (0, 0)),
            pl.BlockSpec((2, _BR, 1), lambda i: (0, i, 0)),
        ],
        out_specs=[
            pl.BlockSpec((2, _BR, FH), lambda i: (0, i, 0)),
            pl.BlockSpec((_BR, 1), lambda i: (i, 0)),
        ],
        out_shape=[
            jax.ShapeDtypeStruct((2, NP, FH), jnp.float32),
            jax.ShapeDtypeStruct((NP, 1), jnp.float32),
        ],
    )(x_p, w1, parts3)


def _mm2_body(p_ref, dinv_ref, b1_ref, w2_ref, t_ref):
    p = p_ref[...]
    dinv = dinv_ref[...]
    h0 = jnp.maximum(p[0] * dinv + b1_ref[:, :FH], 0.0)
    h1 = jnp.maximum(p[1] * dinv + b1_ref[:, FH:], 0.0)
    w2 = w2_ref[...]
    sc = (jnp.dot(h0, w2[:FH], preferred_element_type=jnp.float32)
          + jnp.dot(h1, w2[FH:], preferred_element_type=jnp.float32))
    t_ref[...] = sc * dinv


def _mm2(parts, dinv, b1, w2):
    return pl.pallas_call(
        _mm2_body,
        grid=(NP // _BR,),
        in_specs=[
            pl.BlockSpec((2, _BR, FH), lambda i: (0, i, 0)),
            pl.BlockSpec((_BR, 1), lambda i: (i, 0)),
            pl.BlockSpec((1, F), lambda i: (0, 0)),
            pl.BlockSpec((F, 1), lambda i: (0, 0)),
        ],
        out_specs=pl.BlockSpec((_BR, 1), lambda i: (i, 0)),
        out_shape=jax.ShapeDtypeStruct((NP, 1), jnp.float32),
    )(parts, dinv, b1, w2)


def kernel(x, edge_index, W1, b1, W2, b2):
    src = edge_index[0].astype(jnp.int32)
    dst = edge_index[1].astype(jnp.int32)
    pad = EP - E
    src_p = jnp.concatenate([src, jnp.zeros((pad,), jnp.int32)])
    dst_p = jnp.concatenate([dst, jnp.full((pad,), NP - 1, jnp.int32)])
    srcf = src_p.reshape(32, EPT)
    dst3 = dst_p.reshape(32, NBLK, BK)
    src16 = src_p.reshape(16, NBLKC, BK)
    dstC = dst_p.reshape(16, NBLKC, BK)
    x_p = jnp.pad(x, ((0, NP - N), (0, 0)))

    deg_parts = _deg_kernel(dst3)
    parts3 = deg_parts.reshape(2, NP, 1)
    y_split, dinv = _mm1(x_p, W1, parts3)
    agg_parts = _agg_kernel(y_split, src16, dstC)
    t = _mm2(agg_parts, dinv, b1.reshape(1, F), W2)
    l2_parts = _l2_kernel(t.reshape(NP), srcf, dst3)

    out = (l2_parts[0] + l2_parts[1] + t[:, 0]) * dinv[:, 0] + b2[0]
    return out[:N]
